# Initial kernel scaffold; baseline (speedup 1.0000x reference)
import jax
import jax.numpy as jnp
from jax.experimental import pallas as pl


def kernel(x_m, x_p, W_enc, b_enc, W_dec_m, W_dec_p):
    raise NotImplementedError("write your pallas kernel here")


# speedup vs baseline: 3.2755x; 3.2755x over previous
import functools

import jax
import jax.numpy as jnp
from jax.experimental import pallas as pl
from jax.experimental.pallas import tpu as pltpu

D_IN = 768
D_HIDDEN = 32768
K = 64
B = 64

BH_ENC = 2048
BH_DEC = 2048


def _mono_key(v):
    bits = jax.lax.bitcast_convert_type(v, jnp.int32)
    return jnp.where(bits >= 0, bits, bits ^ jnp.int32(0x7FFFFFFF))


def _enc_body(x_ref, w_ref, b_ref, pre_ref):
    x = x_ref[...]
    w = w_ref[...]
    pre = jax.lax.dot_general(x, w, (((1,), (1,)), ((), ())),
                              preferred_element_type=jnp.float32)
    pre_ref[...] = pre + b_ref[...]


def _thresh_body(pre_ref, t_ref):
    key = _mono_key(pre_ref[...])

    def step(i, t):
        bit = jnp.int32(31) - i
        cand = t + jax.lax.shift_left(jnp.int32(1), bit)
        cnt = jnp.sum((key >= cand).astype(jnp.int32), axis=1, keepdims=True)
        return jnp.where(cnt >= K, cand, t)

    t0 = jnp.full((B, 1), jnp.int32(-2**31), dtype=jnp.int32)
    t = jax.lax.fori_loop(0, 32, step, t0)
    t_ref[...] = jnp.broadcast_to(t, (B, 128))


def _dec_body(pre_ref, t_ref, wm_ref, wp_ref, z_ref, recm_ref, recp_ref,
              accm, accp, *, nh):
    j = pl.program_id(0)
    pre = pre_ref[...]
    t = t_ref[:, :1]
    mask = _mono_key(pre) >= t
    z = jnp.where(mask, jnp.maximum(pre, 0.0), 0.0)
    z_ref[...] = z

    pm = jax.lax.dot_general(z, wm_ref[...], (((1,), (1,)), ((), ())),
                             preferred_element_type=jnp.float32)
    pp = jax.lax.dot_general(z, wp_ref[...], (((1,), (1,)), ((), ())),
                             preferred_element_type=jnp.float32)

    @pl.when(j == 0)
    def _():
        accm[...] = pm
        accp[...] = pp

    @pl.when(j > 0)
    def _():
        accm[...] += pm
        accp[...] += pp

    @pl.when(j == nh - 1)
    def _():
        recm_ref[...] = accm[...]
        recp_ref[...] = accp[...]


def kernel(x_m, x_p, W_enc, b_enc, W_dec_m, W_dec_p):
    x = jnp.concatenate([x_m, x_p], axis=-1)
    b2 = b_enc.reshape(1, D_HIDDEN)

    nh_e = D_HIDDEN // BH_ENC
    pre = pl.pallas_call(
        _enc_body,
        grid=(nh_e,),
        in_specs=[
            pl.BlockSpec((B, 2 * D_IN), lambda j: (0, 0)),
            pl.BlockSpec((BH_ENC, 2 * D_IN), lambda j: (j, 0)),
            pl.BlockSpec((1, BH_ENC), lambda j: (0, j)),
        ],
        out_specs=pl.BlockSpec((B, BH_ENC), lambda j: (0, j)),
        out_shape=jax.ShapeDtypeStruct((B, D_HIDDEN), jnp.float32),
    )(x, W_enc, b2)

    t = pl.pallas_call(
        _thresh_body,
        in_specs=[pl.BlockSpec((B, D_HIDDEN), lambda: (0, 0))],
        out_specs=pl.BlockSpec((B, 128), lambda: (0, 0)),
        out_shape=jax.ShapeDtypeStruct((B, 128), jnp.int32),
    )(pre)

    nh_d = D_HIDDEN // BH_DEC
    z, rec_m, rec_p = pl.pallas_call(
        functools.partial(_dec_body, nh=nh_d),
        grid=(nh_d,),
        in_specs=[
            pl.BlockSpec((B, BH_DEC), lambda j: (0, j)),
            pl.BlockSpec((B, 128), lambda j: (0, 0)),
            pl.BlockSpec((D_IN, BH_DEC), lambda j: (0, j)),
            pl.BlockSpec((D_IN, BH_DEC), lambda j: (0, j)),
        ],
        out_specs=[
            pl.BlockSpec((B, BH_DEC), lambda j: (0, j)),
            pl.BlockSpec((B, D_IN), lambda j: (0, 0)),
            pl.BlockSpec((B, D_IN), lambda j: (0, 0)),
        ],
        out_shape=[
            jax.ShapeDtypeStruct((B, D_HIDDEN), jnp.float32),
            jax.ShapeDtypeStruct((B, D_IN), jnp.float32),
            jax.ShapeDtypeStruct((B, D_IN), jnp.float32),
        ],
        scratch_shapes=[
            pltpu.VMEM((B, D_IN), jnp.float32),
            pltpu.VMEM((B, D_IN), jnp.float32),
        ],
    )(pre, t, W_dec_m, W_dec_p)

    return (rec_m, rec_p, z)
